# baseline (speedup 1.0000x reference)
import jax
import jax.numpy as jnp
from jax import lax
from jax.experimental import pallas as pl

BB = 8


def _body(x_ref, wp_ref, cp_ref, o_ref):
    n = x_ref.shape[1]
    R = BB * n
    x = x_ref[...].reshape(R, 128)
    u = lax.dot_general(
        wp_ref[...], x,
        dimension_numbers=(((1,), (1,)), ((), ())),
        preferred_element_type=jnp.float32,
    )
    t = u.reshape(BB, n) + cp_ref[...]
    p1 = 1.0 / (1.0 + jnp.exp(-t))
    lane = lax.broadcasted_iota(jnp.int32, (BB, 128), 1)
    half = lane // 2
    even = lane % 2 == 0
    for k in range(2 * n // 128):
        src = p1[:, 128 * (k // 2):128 * (k // 2) + 128]
        g = jnp.take_along_axis(src, 64 * (k % 2) + half, axis=1)
        o_ref[:, 128 * k:128 * (k + 1)] = jnp.where(even, 1.0 - g, g)


def kernel(xs, W, b):
    B, N, D = xs.shape
    w = W[1] - W[0]
    c = b[1] - b[0]
    wp = w.reshape(1, D)
    cp = jnp.full((1, N), c, dtype=jnp.float32)
    out = pl.pallas_call(
        _body,
        grid=(B // BB,),
        in_specs=[
            pl.BlockSpec((BB, N, D), lambda i: (i, 0, 0)),
            pl.BlockSpec((1, D), lambda i: (0, 0)),
            pl.BlockSpec((1, N), lambda i: (0, 0)),
        ],
        out_specs=pl.BlockSpec((BB, 2 * N), lambda i: (i, 0)),
        out_shape=jax.ShapeDtypeStruct((B, 2 * N), jnp.float32),
    )(xs, wp, cp)
    return out.reshape(B, N, 2)


# speedup vs baseline: 1.1165x; 1.1165x over previous
import jax
import jax.numpy as jnp
from jax import lax
from jax.experimental import pallas as pl
from jax.experimental.pallas import tpu as pltpu

BB = 8


def _body(x_ref, w_ref, b_ref, o_ref):
    n = x_ref.shape[1]
    R = BB * n
    x = x_ref[...].reshape(R, 128)
    wd = w_ref[1:2, :] - w_ref[0:1, :]
    u = lax.dot_general(
        wd, x,
        dimension_numbers=(((1,), (1,)), ((), ())),
        preferred_element_type=jnp.float32,
    )
    c = b_ref[1] - b_ref[0]
    t = u.reshape(BB, n) + c
    p1 = 1.0 / (1.0 + jnp.exp(-t))
    lane = lax.broadcasted_iota(jnp.int32, (BB, 128), 1)
    half = lane // 2
    even = lane % 2 == 0
    for k in range(2 * n // 128):
        src = p1[:, 128 * (k // 2):128 * (k // 2) + 128]
        g = jnp.take_along_axis(src, 64 * (k % 2) + half, axis=1)
        o_ref[:, 128 * k:128 * (k + 1)] = jnp.where(even, 1.0 - g, g)


def kernel(xs, W, b):
    B, N, D = xs.shape
    out = pl.pallas_call(
        _body,
        grid=(B // BB,),
        in_specs=[
            pl.BlockSpec((BB, N, D), lambda i: (i, 0, 0)),
            pl.BlockSpec((2, D), lambda i: (0, 0)),
            pl.BlockSpec(memory_space=pltpu.SMEM),
        ],
        out_specs=pl.BlockSpec((BB, 2 * N), lambda i: (i, 0)),
        out_shape=jax.ShapeDtypeStruct((B, 2 * N), jnp.float32),
    )(xs, W, b)
    return out.reshape(B, N, 2)
